# baseline (speedup 1.0000x reference)
import jax
import jax.numpy as jnp
from jax.experimental import pallas as pl
from jax.experimental.pallas import tpu as pltpu

_B = 2
_T = 2048
_D = 1024
_F = 4096
_E = 8
_K = 2
_FT = 1024
_NF = _F // _FT


def _router_kernel(x_ref, Wr_ref, br_ref, idx_ref, w_ref, xbf_ref):
    x = x_ref[0]
    xbf_ref[0] = x.astype(jnp.bfloat16)
    pooled = jnp.mean(x, axis=0)[None, :]
    logits = (jnp.dot(pooled, Wr_ref[...], preferred_element_type=jnp.float32)
              + br_ref[...][None, :])
    iota = jax.lax.broadcasted_iota(jnp.int32, (1, _E), 1)
    m1 = jnp.max(logits, axis=1, keepdims=True)
    i1 = jnp.min(jnp.where(logits == m1, iota, _E), axis=1, keepdims=True)
    masked = jnp.where(iota == i1, -jnp.inf, logits)
    m2 = jnp.max(masked, axis=1, keepdims=True)
    i2 = jnp.min(jnp.where(masked == m2, iota, _E), axis=1, keepdims=True)
    d = jnp.exp(m2 - m1)
    w1 = 1.0 / (1.0 + d)
    w2 = d / (1.0 + d)
    idx_ref[0] = jnp.concatenate([i1, i2], axis=1)
    w_ref[0] = jnp.concatenate([w1, w2], axis=1)


def _moe_kernel(idx_ref, x_ref, W1_ref, b1_ref, W2_ref, b2_ref, wts_ref,
                out_ref):
    b = pl.program_id(0)
    k = pl.program_id(1)
    f = pl.program_id(2)
    w = wts_ref[b, k]
    x = x_ref[0]
    h = jnp.dot(x, W1_ref[0].astype(jnp.bfloat16),
                preferred_element_type=jnp.float32)
    h = h + b1_ref[0]
    h = h * (0.5 + 0.5 * jax.lax.erf(h * 0.7071067811865476))
    w2b = W2_ref[0].astype(jnp.bfloat16) * w.astype(jnp.bfloat16)
    contrib = jnp.dot(h.astype(jnp.bfloat16), w2b,
                      preferred_element_type=jnp.float32)
    bias_scale = jnp.where(f == 0, w, 0.0)
    contrib = contrib + bias_scale * b2_ref[0]

    @pl.when(jnp.logical_and(k == 0, f == 0))
    def _init():
        out_ref[0] = contrib

    @pl.when(jnp.logical_or(k != 0, f != 0))
    def _accum():
        out_ref[0] += contrib


def kernel(x, Wr, br, W1, b1, W2, b2):
    top_idx, wts, x_bf = pl.pallas_call(
        _router_kernel,
        grid=(_B,),
        in_specs=[
            pl.BlockSpec((1, _T, _D), lambda b: (b, 0, 0)),
            pl.BlockSpec((_D, _E), lambda b: (0, 0)),
            pl.BlockSpec((_E,), lambda b: (0,)),
        ],
        out_specs=(
            pl.BlockSpec((1, 1, _K), lambda b: (b, 0, 0)),
            pl.BlockSpec((1, 1, _K), lambda b: (b, 0, 0)),
            pl.BlockSpec((1, _T, _D), lambda b: (b, 0, 0)),
        ),
        out_shape=(
            jax.ShapeDtypeStruct((_B, 1, _K), jnp.int32),
            jax.ShapeDtypeStruct((_B, 1, _K), jnp.float32),
            jax.ShapeDtypeStruct((_B, _T, _D), jnp.bfloat16),
        ),
    )(x, Wr, br)
    top_idx = top_idx.reshape(_B, _K)
    wts = wts.reshape(_B, _K)

    grid_spec = pltpu.PrefetchScalarGridSpec(
        num_scalar_prefetch=1,
        grid=(_B, _K, _NF),
        in_specs=[
            pl.BlockSpec((1, _T, _D), lambda b, k, f, idx: (b, 0, 0)),
            pl.BlockSpec((1, _D, _FT), lambda b, k, f, idx: (idx[b, k], 0, f)),
            pl.BlockSpec((1, 1, _FT), lambda b, k, f, idx: (idx[b, k], 0, f)),
            pl.BlockSpec((1, _FT, _D), lambda b, k, f, idx: (idx[b, k], f, 0)),
            pl.BlockSpec((1, 1, _D), lambda b, k, f, idx: (idx[b, k], 0, 0)),
            pl.BlockSpec(memory_space=pltpu.SMEM),
        ],
        out_specs=pl.BlockSpec((1, _T, _D), lambda b, k, f, idx: (b, 0, 0)),
    )
    out = pl.pallas_call(
        _moe_kernel,
        grid_spec=grid_spec,
        out_shape=jax.ShapeDtypeStruct((_B, _T, _D), jnp.float32),
        compiler_params=pltpu.CompilerParams(
            dimension_semantics=("parallel", "arbitrary", "arbitrary"),
            vmem_limit_bytes=112 * 1024 * 1024,
        ),
    )(top_idx, x_bf, W1, b1.reshape(_E, 1, _F), W2, b2.reshape(_E, 1, _D), wts)
    return out
